# baseline (speedup 1.0000x reference)
import functools

import jax
import jax.numpy as jnp
from jax import lax
from jax.experimental import pallas as pl
from jax.experimental.pallas import tpu as pltpu
from jax.experimental.pallas import tpu_sc as plsc

BATCH = 16384
HIST = 50
D = 64
NC = 2
NS = 16
NW = NC * NS
BPW = BATCH // NW
K = 8
NGROUP = BPW // K
NBUF = 2 * K


def _make_gather():
    mesh = plsc.VectorSubcoreMesh(core_axis_name="c", subcore_axis_name="s")

    @functools.partial(
        pl.kernel,
        mesh=mesh,
        compiler_params=pltpu.CompilerParams(use_tc_tiling_on_sc=False),
        out_type=jax.ShapeDtypeStruct((BATCH, HIST, D), jnp.float32),
        scratch_types=[
            pltpu.VMEM((BPW, HIST), jnp.int32),
            pltpu.VMEM((NBUF, HIST, D), jnp.float32),
            pltpu.SemaphoreType.DMA,
            pltpu.SemaphoreType.DMA,
        ],
    )
    def gather_kernel(idx_hbm, table_hbm, out_hbm, idx_v, rows_v, gsem, ssem):
        wid = lax.axis_index("s") * NC + lax.axis_index("c")
        base = wid * BPW
        pltpu.sync_copy(idx_hbm.at[pl.ds(base, BPW)], idx_v)

        def fire_gathers(g, half):
            for b in range(K):
                pltpu.async_copy(
                    table_hbm.at[idx_v.at[g * K + b]],
                    rows_v.at[half * K + b],
                    gsem,
                )

        def drain_gathers():
            for b in range(K):
                pltpu.make_async_copy(
                    table_hbm.at[idx_v.at[0]], rows_v.at[b], gsem
                ).wait()

        def fire_scatters(g, half):
            for b in range(K):
                pltpu.async_copy(
                    rows_v.at[half * K + b],
                    out_hbm.at[base + g * K + b],
                    ssem,
                )

        def drain_scatters():
            for b in range(K):
                pltpu.make_async_copy(
                    rows_v.at[b], out_hbm.at[base], ssem
                ).wait()

        fire_gathers(0, 0)

        def body(g, carry):
            h = lax.rem(g, 2)

            @pl.when(g >= 1)
            def _():
                drain_scatters()

            @pl.when(g + 1 < NGROUP)
            def _():
                fire_gathers(g + 1, 1 - h)

            drain_gathers()
            fire_scatters(g, h)
            return carry

        lax.fori_loop(0, NGROUP, body, 0)
        drain_scatters()

    return gather_kernel


_gather = _make_gather()


def kernel(input_nodes, node_token_emb_weight):
    return _gather(input_nodes, node_token_emb_weight)


# speedup vs baseline: 1.3452x; 1.3452x over previous
import functools

import jax
import jax.numpy as jnp
from jax import lax
from jax.experimental import pallas as pl
from jax.experimental.pallas import tpu as pltpu
from jax.experimental.pallas import tpu_sc as plsc

BATCH = 16384
HIST = 50
D = 64
HP = 56
DP = 128
NC = 2
NS = 16
NW = NC * NS
BPW = BATCH // NW
K = 8
NGROUP = BPW // K
NBUF = 2 * K


def _make_gather():
    mesh = plsc.VectorSubcoreMesh(core_axis_name="c", subcore_axis_name="s")

    @functools.partial(
        pl.kernel,
        mesh=mesh,
        compiler_params=pltpu.CompilerParams(use_tc_tiling_on_sc=False),
        out_type=jax.ShapeDtypeStruct((BATCH, HP, DP), jnp.float32),
        scratch_types=[
            pltpu.VMEM((BPW, HIST), jnp.int32),
            pltpu.VMEM((NBUF, HIST, D), jnp.float32),
            pltpu.SemaphoreType.DMA,
            pltpu.SemaphoreType.DMA,
        ],
    )
    def gather_kernel(idx_hbm, table_hbm, out_hbm, idx_v, rows_v, gsem, ssem):
        wid = lax.axis_index("s") * NC + lax.axis_index("c")
        base = wid * BPW
        pltpu.sync_copy(idx_hbm.at[pl.ds(base, BPW)], idx_v)

        def fire_gathers(g, half):
            for b in range(K):
                pltpu.async_copy(
                    table_hbm.at[idx_v.at[g * K + b]],
                    rows_v.at[half * K + b],
                    gsem,
                )

        def drain_gathers():
            for b in range(K):
                pltpu.make_async_copy(
                    table_hbm.at[idx_v.at[0]], rows_v.at[b], gsem
                ).wait()

        def fire_scatters(g, half):
            for b in range(K):
                pltpu.async_copy(
                    rows_v.at[half * K + b],
                    out_hbm.at[base + g * K + b, pl.ds(0, HIST), pl.ds(0, D)],
                    ssem,
                )

        def drain_scatters():
            for b in range(K):
                pltpu.make_async_copy(
                    rows_v.at[b],
                    out_hbm.at[base, pl.ds(0, HIST), pl.ds(0, D)],
                    ssem,
                ).wait()

        fire_gathers(0, 0)

        def body(g, carry):
            h = lax.rem(g, 2)

            @pl.when(g >= 1)
            def _():
                drain_scatters()

            @pl.when(g + 1 < NGROUP)
            def _():
                fire_gathers(g + 1, 1 - h)

            drain_gathers()
            fire_scatters(g, h)
            return carry

        lax.fori_loop(0, NGROUP, body, 0)
        drain_scatters()

    return gather_kernel


_gather = _make_gather()


def kernel(input_nodes, node_token_emb_weight):
    out = _gather(input_nodes, node_token_emb_weight)
    return out[:, :HIST, :D]


# speedup vs baseline: 1.4441x; 1.0735x over previous
import functools

import jax
import jax.numpy as jnp
from jax import lax
from jax.experimental import pallas as pl
from jax.experimental.pallas import tpu as pltpu
from jax.experimental.pallas import tpu_sc as plsc

BATCH = 16384
HIST = 50
D = 64
NROW = 1000001
NROWP = 1000008
HP = 56
DP = 128
NC = 2
NS = 16
NW = NC * NS
BPW = BATCH // NW
K = 8
NGROUP = BPW // K
NBUF = 2 * K


def _make_gather():
    mesh = plsc.VectorSubcoreMesh(core_axis_name="c", subcore_axis_name="s")

    @functools.partial(
        pl.kernel,
        mesh=mesh,
        compiler_params=pltpu.CompilerParams(use_tc_tiling_on_sc=False),
        out_type=jax.ShapeDtypeStruct((BATCH, HP, DP), jnp.float32),
        scratch_types=[
            pltpu.VMEM((BPW, HIST), jnp.int32),
            pltpu.VMEM((NBUF, HIST, D), jnp.float32),
            pltpu.SemaphoreType.DMA,
            pltpu.SemaphoreType.DMA,
        ],
    )
    def gather_kernel(idx_hbm, table_hbm, out_hbm, idx_v, rows_v, gsem, ssem):
        wid = lax.axis_index("s") * NC + lax.axis_index("c")
        base = wid * BPW
        pltpu.sync_copy(idx_hbm.at[pl.ds(base, BPW)], idx_v)

        def fire_gathers(g, half):
            for b in range(K):
                pltpu.async_copy(
                    table_hbm.at[idx_v.at[g * K + b]],
                    rows_v.at[half * K + b],
                    gsem,
                )

        def drain_gathers():
            for b in range(K):
                pltpu.make_async_copy(
                    table_hbm.at[idx_v.at[0]], rows_v.at[b], gsem
                ).wait()

        def fire_scatters(g, half):
            for b in range(K):
                pltpu.async_copy(
                    rows_v.at[half * K + b],
                    out_hbm.at[base + g * K + b, pl.ds(0, HIST), pl.ds(0, D)],
                    ssem,
                )

        def drain_scatters():
            for b in range(K):
                pltpu.make_async_copy(
                    rows_v.at[b],
                    out_hbm.at[base, pl.ds(0, HIST), pl.ds(0, D)],
                    ssem,
                ).wait()

        fire_gathers(0, 0)

        def body(g, carry):
            h = lax.rem(g, 2)

            @pl.when(g >= 1)
            def _():
                drain_scatters()

            @pl.when(g + 1 < NGROUP)
            def _():
                fire_gathers(g + 1, 1 - h)

            drain_gathers()
            fire_scatters(g, h)
            return carry

        lax.fori_loop(0, NGROUP, body, 0)
        drain_scatters()

    return gather_kernel


_gather = _make_gather()


def kernel(input_nodes, node_token_emb_weight):
    table2 = jnp.pad(
        node_token_emb_weight, ((0, NROWP - NROW), (0, DP - D))
    ).reshape(2 * NROWP, D)
    out = _gather(input_nodes * 2, table2)
    return out[:, :HIST, :D]


# speedup vs baseline: 1.8132x; 1.2556x over previous
import functools

import jax
import jax.numpy as jnp
from jax import lax
from jax.experimental import pallas as pl
from jax.experimental.pallas import tpu as pltpu
from jax.experimental.pallas import tpu_sc as plsc

BATCH = 16384
HIST = 50
D = 64
NROW = 1000001
NROWP = 1000008
HP = 56
DP = 128
NC = 2
NS = 16
NW = NC * NS
BPW = BATCH // NW
K = 8
NGROUP = BPW // K
NBUF = 2 * K


def _make_gather():
    mesh = plsc.VectorSubcoreMesh(core_axis_name="c", subcore_axis_name="s")

    @functools.partial(
        pl.kernel,
        mesh=mesh,
        compiler_params=pltpu.CompilerParams(use_tc_tiling_on_sc=False),
        out_type=jax.ShapeDtypeStruct((BATCH, HP, DP), jnp.float32),
        scratch_types=[
            pltpu.VMEM((BPW, HIST), jnp.int32),
            pltpu.VMEM((NBUF, HIST, D), jnp.float32),
            pltpu.SemaphoreType.DMA,
            pltpu.SemaphoreType.DMA,
        ],
    )
    def gather_kernel(idx_hbm, table_hbm, out_hbm, idx_v, rows_v, gsem, ssem):
        wid = lax.axis_index("s") * NC + lax.axis_index("c")
        base = wid * BPW
        pltpu.sync_copy(idx_hbm.at[pl.ds(base, BPW)], idx_v)

        def fire_gathers(g, half):
            for b in range(K):
                pltpu.async_copy(
                    table_hbm.at[idx_v.at[g * K + b]],
                    rows_v.at[half * K + b],
                    gsem,
                )

        def drain_gathers():
            for b in range(K):
                pltpu.make_async_copy(
                    table_hbm.at[idx_v.at[0]], rows_v.at[b], gsem
                ).wait()

        def fire_scatters(g, half):
            for b in range(K):
                pltpu.async_copy(
                    rows_v.at[half * K + b],
                    out_hbm.at[base + g * K + b, pl.ds(0, HIST), pl.ds(0, D)],
                    ssem,
                )

        def drain_scatters():
            for b in range(K):
                pltpu.make_async_copy(
                    rows_v.at[b],
                    out_hbm.at[base, pl.ds(0, HIST), pl.ds(0, D)],
                    ssem,
                ).wait()

        fire_gathers(0, 0)

        def body(g, carry):
            h = lax.rem(g, 2)

            @pl.when(g >= 1)
            def _():
                drain_scatters()

            @pl.when(g + 1 < NGROUP)
            def _():
                fire_gathers(g + 1, 1 - h)

            drain_gathers()
            fire_scatters(g, h)
            return carry

        lax.fori_loop(0, NGROUP, body, 0)
        drain_scatters()

    return gather_kernel


_gather = _make_gather()

CBLK = 4096
NBLK = -(-NROW // CBLK)


def _tformat_kernel(tin_ref, out_ref):
    x = tin_ref[...]
    out_ref[...] = jnp.concatenate(
        [x.T, jnp.zeros((CBLK, DP - D), jnp.float32)], axis=1
    )


_tformat = pl.pallas_call(
    _tformat_kernel,
    grid=(NBLK,),
    in_specs=[pl.BlockSpec((D, CBLK), lambda i: (0, i))],
    out_specs=pl.BlockSpec((CBLK, DP), lambda i: (i, 0)),
    out_shape=jax.ShapeDtypeStruct((NROWP, DP), jnp.float32),
)


def kernel(input_nodes, node_token_emb_weight):
    table2 = _tformat(node_token_emb_weight.T).reshape(2 * NROWP, D)
    out = _gather(input_nodes * 2, table2)
    return out[:, :HIST, :D]


# speedup vs baseline: 2.0389x; 1.1245x over previous
import functools

import jax
import jax.numpy as jnp
from jax import lax
from jax.experimental import pallas as pl
from jax.experimental.pallas import tpu as pltpu
from jax.experimental.pallas import tpu_sc as plsc

BATCH = 16384
HIST = 50
D = 64
NROW = 1000001
NROWP = 1000008
HP = 56
DP = 128
NC = 2
NS = 16
NW = NC * NS
BPW = BATCH // NW
K = 8
NGROUP = BPW // K
NBUF = 2 * K


def _make_gather():
    mesh = plsc.VectorSubcoreMesh(core_axis_name="c", subcore_axis_name="s")

    @functools.partial(
        pl.kernel,
        mesh=mesh,
        compiler_params=pltpu.CompilerParams(use_tc_tiling_on_sc=False),
        out_type=jax.ShapeDtypeStruct((BATCH, HP, DP), jnp.float32),
        scratch_types=[
            pltpu.VMEM((BPW, HIST), jnp.int32),
            pltpu.VMEM((NBUF, HIST, D), jnp.float32),
            pltpu.SemaphoreType.DMA,
            pltpu.SemaphoreType.DMA,
        ],
    )
    def gather_kernel(idx_hbm, table_hbm, out_hbm, idx_v, rows_v, gsem, ssem):
        wid = lax.axis_index("s") * NC + lax.axis_index("c")
        base = wid * BPW
        pltpu.sync_copy(idx_hbm.at[pl.ds(base, BPW)], idx_v)

        def fire_gathers(g, half):
            for b in range(K):
                pltpu.async_copy(
                    table_hbm.at[idx_v.at[g * K + b]],
                    rows_v.at[half * K + b],
                    gsem,
                )

        def drain_gathers():
            for b in range(K):
                pltpu.make_async_copy(
                    table_hbm.at[idx_v.at[0]], rows_v.at[b], gsem
                ).wait()

        def fire_scatters(g, half):
            for b in range(K):
                pltpu.async_copy(
                    rows_v.at[half * K + b],
                    out_hbm.at[base + g * K + b, pl.ds(0, HIST), pl.ds(0, D)],
                    ssem,
                )

        def drain_scatters():
            for b in range(K):
                pltpu.make_async_copy(
                    rows_v.at[b],
                    out_hbm.at[base, pl.ds(0, HIST), pl.ds(0, D)],
                    ssem,
                ).wait()

        fire_gathers(0, 0)

        def body(g, carry):
            h = lax.rem(g, 2)

            @pl.when(g >= 1)
            def _():
                drain_scatters()

            @pl.when(g + 1 < NGROUP)
            def _():
                fire_gathers(g + 1, 1 - h)

            drain_gathers()
            fire_scatters(g, h)
            return carry

        lax.fori_loop(0, NGROUP, body, 0)
        drain_scatters()

    return gather_kernel


_gather = _make_gather()

CBLK = 8192
NBLK = -(-NROW // CBLK)


def _tformat_kernel(tin_ref, out_ref):
    x = tin_ref[...]
    out_ref[...] = jnp.concatenate(
        [x.T, jnp.zeros((CBLK, DP - D), jnp.float32)], axis=1
    )


_tformat = pl.pallas_call(
    _tformat_kernel,
    grid=(NBLK,),
    in_specs=[pl.BlockSpec((D, CBLK), lambda i: (0, i))],
    out_specs=pl.BlockSpec((CBLK, DP), lambda i: (i, 0)),
    out_shape=jax.ShapeDtypeStruct((NROWP, DP), jnp.float32),
)


def kernel(input_nodes, node_token_emb_weight):
    table2 = _tformat(node_token_emb_weight.T).reshape(2 * NROWP, D)
    out = _gather(input_nodes * 2, table2)
    return out[:, :HIST, :D]


# speedup vs baseline: 2.0888x; 1.0244x over previous
import functools

import jax
import jax.numpy as jnp
from jax import lax
from jax.experimental import pallas as pl
from jax.experimental.pallas import tpu as pltpu
from jax.experimental.pallas import tpu_sc as plsc

BATCH = 16384
HIST = 50
D = 64
NROW = 1000001
NROWP = 1000008
HP = 56
DP = 128
NC = 2
NS = 16
NW = NC * NS
BPW = BATCH // NW
K = 8
NGROUP = BPW // K
NBUF = 2 * K


def _make_gather():
    mesh = plsc.VectorSubcoreMesh(core_axis_name="c", subcore_axis_name="s")

    @functools.partial(
        pl.kernel,
        mesh=mesh,
        compiler_params=pltpu.CompilerParams(use_tc_tiling_on_sc=False),
        out_type=jax.ShapeDtypeStruct((BATCH, HP, DP), jnp.float32),
        scratch_types=[
            pltpu.VMEM((BPW, HIST), jnp.int32),
            pltpu.VMEM((NBUF, HIST, D), jnp.float32),
            pltpu.SemaphoreType.DMA,
            pltpu.SemaphoreType.DMA,
        ],
    )
    def gather_kernel(idx_hbm, table_hbm, out_hbm, idx_v, rows_v, gsem, ssem):
        wid = lax.axis_index("s") * NC + lax.axis_index("c")
        base = wid * BPW
        pltpu.sync_copy(idx_hbm.at[pl.ds(base, BPW)], idx_v)

        def fire_gathers(g, half):
            for b in range(K):
                pltpu.async_copy(
                    table_hbm.at[idx_v.at[g * K + b]],
                    rows_v.at[half * K + b],
                    gsem,
                )

        def drain_gathers():
            for b in range(K):
                pltpu.make_async_copy(
                    table_hbm.at[idx_v.at[0]], rows_v.at[b], gsem
                ).wait()

        def fire_scatters(g, half):
            for b in range(K):
                pltpu.async_copy(
                    rows_v.at[half * K + b],
                    out_hbm.at[base + g * K + b, pl.ds(0, HIST), pl.ds(0, D)],
                    ssem,
                )

        def drain_scatters():
            for b in range(K):
                pltpu.make_async_copy(
                    rows_v.at[b],
                    out_hbm.at[base, pl.ds(0, HIST), pl.ds(0, D)],
                    ssem,
                ).wait()

        fire_gathers(0, 0)

        def body(g, carry):
            h = lax.rem(g, 2)

            @pl.when(g >= 1)
            def _():
                drain_scatters()

            @pl.when(g + 1 < NGROUP)
            def _():
                fire_gathers(g + 1, 1 - h)

            drain_gathers()
            fire_scatters(g, h)
            return carry

        lax.fori_loop(0, NGROUP, body, 0)
        drain_scatters()

    return gather_kernel


_gather = _make_gather()

CBLK = 16384
NBLK = -(-NROW // CBLK)


def _tformat_kernel(tin_ref, out_ref):
    x = tin_ref[...]
    out_ref[...] = jnp.concatenate(
        [x.T, jnp.zeros((CBLK, DP - D), jnp.float32)], axis=1
    )


_tformat = pl.pallas_call(
    _tformat_kernel,
    grid=(NBLK,),
    in_specs=[pl.BlockSpec((D, CBLK), lambda i: (0, i))],
    out_specs=pl.BlockSpec((CBLK, DP), lambda i: (i, 0)),
    out_shape=jax.ShapeDtypeStruct((NROWP, DP), jnp.float32),
)


def kernel(input_nodes, node_token_emb_weight):
    table2 = _tformat(node_token_emb_weight.T).reshape(2 * NROWP, D)
    out = _gather(input_nodes * 2, table2)
    return out[:, :HIST, :D]


# speedup vs baseline: 2.1087x; 1.0096x over previous
import functools

import jax
import jax.numpy as jnp
from jax import lax
from jax.experimental import pallas as pl
from jax.experimental.pallas import tpu as pltpu
from jax.experimental.pallas import tpu_sc as plsc

BATCH = 16384
HIST = 50
D = 64
NROW = 1000001
NROWP = 1000008
HP = 56
DP = 128
NC = 2
NS = 16
NW = NC * NS
BPW = BATCH // NW
K = 8
NGROUP = BPW // K
NBUF = 2 * K


def _make_gather():
    mesh = plsc.VectorSubcoreMesh(core_axis_name="c", subcore_axis_name="s")

    @functools.partial(
        pl.kernel,
        mesh=mesh,
        compiler_params=pltpu.CompilerParams(use_tc_tiling_on_sc=False),
        out_type=jax.ShapeDtypeStruct((BATCH, HP, DP), jnp.float32),
        scratch_types=[
            pltpu.VMEM((BPW, HIST), jnp.int32),
            pltpu.VMEM((NBUF, HIST, D), jnp.float32),
            pltpu.SemaphoreType.DMA,
            pltpu.SemaphoreType.DMA,
        ],
    )
    def gather_kernel(idx_hbm, table_hbm, out_hbm, idx_v, rows_v, gsem, ssem):
        wid = lax.axis_index("s") * NC + lax.axis_index("c")
        base = wid * BPW
        pltpu.sync_copy(idx_hbm.at[pl.ds(base, BPW)], idx_v)

        def fire_gathers(g, half):
            for b in range(K):
                pltpu.async_copy(
                    table_hbm.at[idx_v.at[g * K + b]],
                    rows_v.at[half * K + b],
                    gsem,
                )

        def drain_gathers():
            for b in range(K):
                pltpu.make_async_copy(
                    table_hbm.at[idx_v.at[0]], rows_v.at[b], gsem
                ).wait()

        def fire_scatters(g, half):
            for b in range(K):
                pltpu.async_copy(
                    rows_v.at[half * K + b],
                    out_hbm.at[base + g * K + b, pl.ds(0, HIST), pl.ds(0, D)],
                    ssem,
                )

        def drain_scatters():
            for b in range(K):
                pltpu.make_async_copy(
                    rows_v.at[b],
                    out_hbm.at[base, pl.ds(0, HIST), pl.ds(0, D)],
                    ssem,
                ).wait()

        fire_gathers(0, 0)

        def body(g, carry):
            h = lax.rem(g, 2)

            @pl.when(g >= 1)
            def _():
                drain_scatters()

            @pl.when(g + 1 < NGROUP)
            def _():
                fire_gathers(g + 1, 1 - h)

            drain_gathers()
            fire_scatters(g, h)
            return carry

        lax.fori_loop(0, NGROUP, body, 0)
        drain_scatters()

    return gather_kernel


_gather = _make_gather()

CBLK = 32768
NBLK = -(-NROW // CBLK)


def _tformat_kernel(tin_ref, out_ref):
    x = tin_ref[...]
    out_ref[...] = jnp.concatenate(
        [x.T, jnp.zeros((CBLK, DP - D), jnp.float32)], axis=1
    )


_tformat = pl.pallas_call(
    _tformat_kernel,
    grid=(NBLK,),
    in_specs=[pl.BlockSpec((D, CBLK), lambda i: (0, i))],
    out_specs=pl.BlockSpec((CBLK, DP), lambda i: (i, 0)),
    out_shape=jax.ShapeDtypeStruct((NROWP, DP), jnp.float32),
)


def kernel(input_nodes, node_token_emb_weight):
    table2 = _tformat(node_token_emb_weight.T).reshape(2 * NROWP, D)
    out = _gather(input_nodes * 2, table2)
    return out[:, :HIST, :D]
